# baseline (speedup 1.0000x reference)
import functools

import jax
import jax.numpy as jnp
from jax.experimental import pallas as pl


def _dot(a, b, precision=None):
    return jax.lax.dot_general(
        a, b, (((1,), (0,)), ((), ())),
        preferred_element_type=jnp.float32, precision=precision)


def _tapconv(inputs, streams, Tout, cond=None, stats_groups=None):
    B = inputs[0].shape[0]
    Cout = streams[0]['taps'][0][2].shape[0]
    n_streams = len(streams)
    groups = stats_groups or []

    weights = [w for s in streams for (_i, _s, w) in s['taps']]
    biases = [s['bias'][:, None] for s in streams]
    cond3 = None if cond is None else cond[:, :, None]

    def body(*refs):
        k = 0
        in_refs = refs[k:k + len(inputs)]; k += len(inputs)
        w_refs = refs[k:k + len(weights)]; k += len(weights)
        b_refs = refs[k:k + n_streams]; k += n_streams
        if cond3 is not None:
            c_ref = refs[k]; k += 1
        out_refs = refs[k:k + n_streams]; k += n_streams
        st_refs = refs[k:k + len(groups)]; k += len(groups)

        b = pl.program_id(0)
        full = [r[0].astype(jnp.bfloat16) for r in in_refs]
        slabs = {}
        for s in streams:
            for (ii, sh, _w) in s['taps']:
                if (ii, sh) not in slabs:
                    slabs[(ii, sh)] = jax.lax.slice(
                        full[ii], (0, sh), (full[ii].shape[0], sh + Tout))

        wi = 0
        accs = []
        for si, s in enumerate(streams):
            acc = None
            for (ii, sh, _w) in s['taps']:
                part = _dot(w_refs[wi][...], slabs[(ii, sh)])
                wi += 1
                acc = part if acc is None else acc + part
            acc = acc + b_refs[si][...]
            if cond3 is not None:
                acc = acc + c_ref[0]
            out_refs[si][0] = acc
            accs.append(acc)

        if groups:
            first = b == 0
            for gi, grp in enumerate(groups):
                s1 = None
                s2 = None
                for si in grp:
                    a = accs[si]
                    p1 = jnp.sum(a, axis=1, keepdims=True)
                    p2 = jnp.sum(a * a, axis=1, keepdims=True)
                    s1 = p1 if s1 is None else s1 + p1
                    s2 = p2 if s2 is None else s2 + p2
                upd = jnp.concatenate([s1, s2], axis=1)

                @pl.when(first)
                def _init(ref=st_refs[gi]):
                    ref[...] = jnp.zeros_like(ref)
                st_refs[gi][...] += upd

    in_specs = (
        [pl.BlockSpec((1, a.shape[1], a.shape[2]), lambda b: (b, 0, 0)) for a in inputs]
        + [pl.BlockSpec(w.shape, lambda b: (0, 0)) for w in weights]
        + [pl.BlockSpec((Cout, 1), lambda b: (0, 0)) for _ in biases]
        + ([pl.BlockSpec((1, Cout, 1), lambda b: (b, 0, 0))] if cond3 is not None else [])
    )
    out_shape = (
        [jax.ShapeDtypeStruct((B, Cout, Tout), jnp.float32) for _ in streams]
        + [jax.ShapeDtypeStruct((Cout, 2), jnp.float32) for _ in groups]
    )
    out_specs = (
        [pl.BlockSpec((1, Cout, Tout), lambda b: (b, 0, 0)) for _ in streams]
        + [pl.BlockSpec((Cout, 2), lambda b: (0, 0)) for _ in groups]
    )
    args = list(inputs) + weights + biases + ([cond3] if cond3 is not None else [])
    res = pl.pallas_call(
        body,
        grid=(B,),
        in_specs=in_specs,
        out_specs=out_specs,
        out_shape=out_shape,
    )(*args)
    outs = list(res[:n_streams])
    stats = list(res[n_streams:])
    return outs, stats


def _bn_affine(stats, n, g, b):
    s1 = stats[:, 0]
    s2 = stats[:, 1]
    m = s1 / n
    v = s2 / n - m * m
    sc = g * jax.lax.rsqrt(v + 1e-5)
    return sc, b - m * sc


def _bnact_body(c_ref, g_ref, a_ref, o_ref):
    cc = c_ref[0] * a_ref[:, 0:1] + a_ref[:, 1:2]
    gg = g_ref[0] * a_ref[:, 2:3] + a_ref[:, 3:4]
    o_ref[0] = jnp.tanh(cc) * jax.nn.sigmoid(gg)


def _bnact(c, g, a):
    B, C, T = c.shape
    return pl.pallas_call(
        _bnact_body,
        grid=(B,),
        in_specs=[
            pl.BlockSpec((1, C, T), lambda b: (b, 0, 0)),
            pl.BlockSpec((1, C, T), lambda b: (b, 0, 0)),
            pl.BlockSpec((C, 4), lambda b: (0, 0)),
        ],
        out_specs=pl.BlockSpec((1, C, T), lambda b: (b, 0, 0)),
        out_shape=jax.ShapeDtypeStruct((B, C, T), jnp.float32),
    )(c, g, a)


def _vq_body(z_ref, a_ref, m2cb_ref, cb2_ref, cbt_ref, o_ref):
    hi = jax.lax.Precision.HIGHEST
    z = z_ref[0] * a_ref[:, 0:1] + a_ref[:, 1:2]
    d = _dot(m2cb_ref[...], z, precision=hi) + cb2_ref[...]
    mn = jnp.min(d, axis=0, keepdims=True)
    io = jax.lax.broadcasted_iota(jnp.int32, d.shape, 0)
    idx = jnp.min(jnp.where(d == mn, io, d.shape[0]), axis=0, keepdims=True)
    oh = (io == idx).astype(jnp.float32)
    o_ref[0] = _dot(cbt_ref[...], oh, precision=hi)


def _vq(z_raw, a, codebook):
    B, D, T = z_raw.shape
    n = codebook.shape[0]
    m2cb = -2.0 * codebook
    cb2 = jnp.sum(codebook * codebook, axis=1)[:, None]
    return pl.pallas_call(
        _vq_body,
        grid=(B,),
        in_specs=[
            pl.BlockSpec((1, D, T), lambda b: (b, 0, 0)),
            pl.BlockSpec((D, 2), lambda b: (0, 0)),
            pl.BlockSpec((n, D), lambda b: (0, 0)),
            pl.BlockSpec((n, 1), lambda b: (0, 0)),
            pl.BlockSpec((D, n), lambda b: (0, 0)),
        ],
        out_specs=pl.BlockSpec((1, D, T), lambda b: (b, 0, 0)),
        out_shape=jax.ShapeDtypeStruct((B, D, T), jnp.float32),
    )(z_raw, a, m2cb, cb2, codebook.T)


def kernel(input, speaker, params):
    x = input
    B = x.shape[0]
    bf = jnp.bfloat16

    for lp in params['encoder']:
        Tin = x.shape[2]
        Tout = (Tin - 4) // 2 + 1
        xe = x[:, :, 0::2]
        xo = x[:, :, 1::2]
        wc = lp['conv_w'].astype(bf)
        wg = lp['gate_w'].astype(bf)
        streams = [
            {'taps': [(0, 0, wc[:, :, 0]), (1, 0, wc[:, :, 1]),
                      (0, 1, wc[:, :, 2]), (1, 1, wc[:, :, 3])],
             'bias': lp['conv_b']},
            {'taps': [(0, 0, wg[:, :, 0]), (1, 0, wg[:, :, 1]),
                      (0, 1, wg[:, :, 2]), (1, 1, wg[:, :, 3])],
             'bias': lp['gate_b']},
        ]
        (c_raw, g_raw), (st_c, st_g) = _tapconv(
            [xe, xo], streams, Tout, stats_groups=[[0], [1]])
        n = B * Tout
        scc, shc = _bn_affine(st_c, n, lp['conv_bn_g'], lp['conv_bn_b'])
        scg, shg = _bn_affine(st_g, n, lp['gate_bn_g'], lp['gate_bn_b'])
        x = _bnact(c_raw, g_raw, jnp.stack([scc, shc, scg, shg], axis=1))

    T = x.shape[2]
    wl = params['latent_w'][:, :, 0].astype(bf)
    (z_raw,), (st_z,) = _tapconv(
        [x], [{'taps': [(0, 0, wl)], 'bias': params['latent_b']}],
        T, stats_groups=[[0]])
    scz, shz = _bn_affine(st_z, B * T, params['latent_bn_g'], params['latent_bn_b'])
    x = _vq(z_raw, jnp.stack([scz, shz], axis=1), params['codebook'])

    h = jnp.take(params['speaker_emb'], speaker, axis=0)
    for lp in params['generator']:
        Tin = x.shape[2]
        cond = h @ lp['cond_w'].T + lp['cond_b']
        xpad = jnp.pad(x, ((0, 0), (0, 0), (1, 1)))
        wc = jnp.transpose(lp['conv_w'], (1, 0, 2))[:, :, ::-1].astype(bf)
        wg = jnp.transpose(lp['gate_w'], (1, 0, 2))[:, :, ::-1].astype(bf)
        streams = [
            {'taps': [(0, 0, wc[:, :, 0]), (0, 1, wc[:, :, 2])], 'bias': lp['conv_b']},
            {'taps': [(0, 1, wc[:, :, 1]), (0, 2, wc[:, :, 3])], 'bias': lp['conv_b']},
            {'taps': [(0, 0, wg[:, :, 0]), (0, 1, wg[:, :, 2])], 'bias': lp['gate_b']},
            {'taps': [(0, 1, wg[:, :, 1]), (0, 2, wg[:, :, 3])], 'bias': lp['gate_b']},
        ]
        (ce, co, ge, go), (st_c, st_g) = _tapconv(
            [xpad], streams, Tin, cond=cond, stats_groups=[[0, 1], [2, 3]])
        n = B * 2 * Tin
        scc, shc = _bn_affine(st_c, n, lp['conv_bn_g'], lp['conv_bn_b'])
        scg, shg = _bn_affine(st_g, n, lp['gate_bn_g'], lp['gate_bn_b'])
        a = jnp.stack([scc, shc, scg, shg], axis=1)
        xe = _bnact(ce, ge, a)
        xo = _bnact(co, go, a)
        Cg = xe.shape[1]
        x = jnp.stack([xe, xo], axis=-1).reshape(B, Cg, 2 * Tin)

    T = x.shape[2]
    wp = params['pre_w'][:, :, 0].astype(bf)
    (pre_raw,), (st_p,) = _tapconv(
        [x], [{'taps': [(0, 0, wp)], 'bias': params['pre_b']}],
        T, stats_groups=[[0]])
    scp, shp = _bn_affine(st_p, B * T, params['pre_bn_g'], params['pre_bn_b'])
    wlog = params['logit_w'][:, :, 0]
    wf = (wlog * scp[None, :]).astype(bf)
    bfold = wlog @ shp + params['logit_b']
    (logits,), _ = _tapconv(
        [pre_raw], [{'taps': [(0, 0, wf)], 'bias': bfold}], T)
    return logits
